# baseline (speedup 1.0000x reference)
import functools

import jax
import jax.numpy as jnp
import numpy as np
from jax import lax
from jax.experimental import pallas as pl
from jax.experimental.pallas import tpu as pltpu
from jax.experimental.pallas import tpu_sc as plsc

REP = 128
NE = 320000
NCYC = 5000
SIZES = (3, 4, 5, 6, 7, 8)
TOTC = NCYC * len(SIZES)
EPS = 1e-5


def _cyc_pass1_body(cr_ref, pooled_ref, a_ref, m_ref, stat_ref):
    i = pl.program_id(0)
    y1 = (jnp.dot(cr_ref[...], a_ref[...], preferred_element_type=jnp.float32)
          + jnp.dot(pooled_ref[...], m_ref[0], preferred_element_type=jnp.float32))

    @pl.when(i == 0)
    def _():
        stat_ref[...] = jnp.zeros_like(stat_ref)

    stat_ref[0, :] += jnp.sum(y1, axis=0)
    stat_ref[1, :] += jnp.sum(y1 * y1, axis=0)


def _cyc_pass2_body(cr_ref, pooled_ref, a_ref, m_ref, sc1_ref, w2_ref,
                    y2_ref, stat_ref):
    i = pl.program_id(0)
    y1 = (jnp.dot(cr_ref[...], a_ref[...], preferred_element_type=jnp.float32)
          + jnp.dot(pooled_ref[...], m_ref[0], preferred_element_type=jnp.float32))
    h = jnp.maximum(y1 * sc1_ref[0, :] + sc1_ref[1, :], 0.0)
    y2 = jnp.dot(h, w2_ref[...], preferred_element_type=jnp.float32)
    y2_ref[...] = y2

    @pl.when(i == 0)
    def _():
        stat_ref[...] = jnp.zeros_like(stat_ref)

    stat_ref[0, :] += jnp.sum(y2, axis=0)
    stat_ref[1, :] += jnp.sum(y2 * y2, axis=0)


def _norm_body(y_ref, sc_ref, out_ref):
    out_ref[...] = jnp.maximum(y_ref[...] * sc_ref[0, :] + sc_ref[1, :], 0.0)


def _edge_pass1_body(er_ref, c2e_ref, w1a_ref, w1b_ref, stat_ref):
    i = pl.program_id(0)
    y1 = (jnp.dot(er_ref[...], w1a_ref[...], preferred_element_type=jnp.float32)
          + jnp.dot(c2e_ref[...], w1b_ref[...], preferred_element_type=jnp.float32))

    @pl.when(i == 0)
    def _():
        stat_ref[...] = jnp.zeros_like(stat_ref)

    stat_ref[0, :] += jnp.sum(y1, axis=0)
    stat_ref[1, :] += jnp.sum(y1 * y1, axis=0)


def _edge_pass2_body(er_ref, c2e_ref, w1a_ref, w1b_ref, sc1_ref, w2_ref,
                     y2_ref, stat_ref):
    i = pl.program_id(0)
    y1 = (jnp.dot(er_ref[...], w1a_ref[...], preferred_element_type=jnp.float32)
          + jnp.dot(c2e_ref[...], w1b_ref[...], preferred_element_type=jnp.float32))
    h = jnp.maximum(y1 * sc1_ref[0, :] + sc1_ref[1, :], 0.0)
    y2 = jnp.dot(h, w2_ref[...], preferred_element_type=jnp.float32)
    y2_ref[...] = y2

    @pl.when(i == 0)
    def _():
        stat_ref[...] = jnp.zeros_like(stat_ref)

    stat_ref[0, :] += jnp.sum(y2, axis=0)
    stat_ref[1, :] += jnp.sum(y2 * y2, axis=0)


def _scale_shift(stats, n, g, b):
    mean = stats[0] / n
    var = stats[1] / n - mean * mean
    scale = g / jnp.sqrt(var + EPS)
    shift = b - mean * scale
    return jnp.stack([scale, shift])


_CB = 1000
_CG = TOTC // _CB
_EB = 2000
_EG = NE // _EB


def _row_spec(blk, width):
    return pl.BlockSpec((blk, width), lambda i: (i, 0))


def _full_spec(shape):
    return pl.BlockSpec(shape, lambda i: tuple(0 for _ in shape))


def _cycle_mlp(cycle_rep, pooled, A, Ms, cyc_g1, cyc_b1, cyc_W2, cyc_g2, cyc_b2):
    m_spec = pl.BlockSpec((1, REP, 2 * REP), lambda i: (i // (NCYC // _CB), 0, 0))
    stats1 = pl.pallas_call(
        _cyc_pass1_body,
        grid=(_CG,),
        in_specs=[_row_spec(_CB, REP), _row_spec(_CB, REP),
                  _full_spec((REP, 2 * REP)), m_spec],
        out_specs=_full_spec((2, 2 * REP)),
        out_shape=jax.ShapeDtypeStruct((2, 2 * REP), jnp.float32),
    )(cycle_rep, pooled, A, Ms)
    sc1 = _scale_shift(stats1, TOTC, cyc_g1, cyc_b1)

    y2, stats2 = pl.pallas_call(
        _cyc_pass2_body,
        grid=(_CG,),
        in_specs=[_row_spec(_CB, REP), _row_spec(_CB, REP),
                  _full_spec((REP, 2 * REP)), m_spec,
                  _full_spec((2, 2 * REP)), _full_spec((2 * REP, REP))],
        out_specs=[_row_spec(_CB, REP), _full_spec((2, REP))],
        out_shape=[jax.ShapeDtypeStruct((TOTC, REP), jnp.float32),
                   jax.ShapeDtypeStruct((2, REP), jnp.float32)],
    )(cycle_rep, pooled, A, Ms, sc1, cyc_W2)
    sc2 = _scale_shift(stats2, TOTC, cyc_g2, cyc_b2)

    cycle_out = pl.pallas_call(
        _norm_body,
        grid=(_CG,),
        in_specs=[_row_spec(_CB, REP), _full_spec((2, REP))],
        out_specs=_row_spec(_CB, REP),
        out_shape=jax.ShapeDtypeStruct((TOTC, REP), jnp.float32),
    )(y2, sc2)
    return cycle_out


def _edge_mlp(edge_rep, c2e, W1a, W1b, edge_g1, edge_b1, edge_W2, edge_g2, edge_b2):
    stats1 = pl.pallas_call(
        _edge_pass1_body,
        grid=(_EG,),
        in_specs=[_row_spec(_EB, REP), _row_spec(_EB, REP),
                  _full_spec((REP, 2 * REP)), _full_spec((REP, 2 * REP))],
        out_specs=_full_spec((2, 2 * REP)),
        out_shape=jax.ShapeDtypeStruct((2, 2 * REP), jnp.float32),
    )(edge_rep, c2e, W1a, W1b)
    sc1 = _scale_shift(stats1, NE, edge_g1, edge_b1)

    y2, stats2 = pl.pallas_call(
        _edge_pass2_body,
        grid=(_EG,),
        in_specs=[_row_spec(_EB, REP), _row_spec(_EB, REP),
                  _full_spec((REP, 2 * REP)), _full_spec((REP, 2 * REP)),
                  _full_spec((2, 2 * REP)), _full_spec((2 * REP, REP))],
        out_specs=[_row_spec(_EB, REP), _full_spec((2, REP))],
        out_shape=[jax.ShapeDtypeStruct((NE, REP), jnp.float32),
                   jax.ShapeDtypeStruct((2, REP), jnp.float32)],
    )(edge_rep, c2e, W1a, W1b, sc1, edge_W2)
    sc2 = _scale_shift(stats2, NE, edge_g2, edge_b2)

    edge_out = pl.pallas_call(
        _norm_body,
        grid=(_EG,),
        in_specs=[_row_spec(_EB, REP), _full_spec((2, REP))],
        out_specs=_row_spec(_EB, REP),
        out_shape=jax.ShapeDtypeStruct((NE, REP), jnp.float32),
    )(y2, sc2)
    return edge_out


def _gather_pooled(edge_rep, idxs):
    per_size = [jnp.take(edge_rep, idx, axis=0).sum(axis=1) for idx in idxs]
    return jnp.concatenate(per_size, axis=0)


_NSC = 2
_NT = 16
_ITEMS = sum(NCYC * s for s in SIZES)
_CS = 8192
_NCH = 40
_NEP = _NCH * _CS
_NPASS = _NCH // _NSC
_TRASH = 8
_DR = _CS // _NT
_ZB = 64
_GPT = (((_ITEMS + _NT - 1) // _NT + 255) // 256) * 16
_IPW = _GPT * 16
_NPADI = _NT * _IPW


def _sc_scatter_body(co_hbm, eid_hbm, rid_hbm, lane_hbm, flag_hbm, c2e_hbm,
                     acc, e_v, r_v, f_v, lbuf, g1d, rows_v, zbuf, lane_v, sem):
    cid = lax.axis_index("c")
    sid = lax.axis_index("s")

    pltpu.sync_copy(lane_hbm, lane_v)
    pltpu.sync_copy(eid_hbm.at[pl.ds(sid * _IPW, _IPW)], e_v)
    pltpu.sync_copy(rid_hbm.at[pl.ds(sid * _IPW, _IPW)], r_v)

    def _zr(i, _):
        zbuf[i // 8, pl.ds((i % 8) * 16, 16)] = jnp.zeros((16,), jnp.float32)
        return 0
    lax.fori_loop(0, _ZB * 8, _zr, 0)
    lane = lane_v[pl.ds(0, 16)]

    def _pass(p, _):
        q = _NSC * p + cid
        base = q * _CS

        for k in range(_DR // _ZB):
            pltpu.sync_copy(zbuf, acc.at[pl.ds(sid * _DR + k * _ZB, _ZB)])
        pltpu.sync_copy(
            flag_hbm.at[pl.ds((sid * _NCH + q) * _GPT, _GPT)], f_v)
        plsc.subcore_barrier()

        def _scan(gg, _):
            fv = f_v[pl.ds(gg * 16, 16)]
            for t in range(16):
                off = gg * 256 + t * 16
                ev = e_v[pl.ds(off, 16)]
                ls = ev - base
                m = (ls >= 0) & (ls < _CS)
                lbuf[pl.ds(0, 16)] = jnp.where(
                    m, ls, _CS + (lane & (_TRASH - 1)))
                g1d[pl.ds(0, 16)] = r_v[pl.ds(off, 16)]

                @pl.when(fv[t] > 0)
                def _():
                    pltpu.sync_copy(co_hbm.at[g1d], rows_v)
                    pltpu.sync_copy(rows_v, acc.at[lbuf], add=True)
            return 0
        lax.fori_loop(0, _GPT // 16, _scan, 0)

        plsc.subcore_barrier()
        pltpu.sync_copy(acc.at[pl.ds(sid * _DR, _DR)],
                        c2e_hbm.at[pl.ds(base + sid * _DR, _DR)])
        return 0

    lax.fori_loop(0, _NPASS, _pass, 0)


def _sc_scatter(cycle_out, eid, rid, lanes, flags):
    mesh = plsc.VectorSubcoreMesh(core_axis_name="c", subcore_axis_name="s",
                                  num_cores=_NSC, num_subcores=_NT)
    f = pl.kernel(
        _sc_scatter_body,
        out_type=jax.ShapeDtypeStruct((_NEP, REP), jnp.float32),

        mesh=mesh,
        scratch_types=[
            pltpu.VMEM_SHARED((_CS + _TRASH, REP), jnp.float32),
            pltpu.VMEM((_IPW,), jnp.int32),
            pltpu.VMEM((_IPW,), jnp.int32),
            pltpu.VMEM((_GPT,), jnp.int32),
            pltpu.VMEM((16,), jnp.int32),
            pltpu.VMEM((16,), jnp.int32),
            pltpu.VMEM((16, REP), jnp.float32),
            pltpu.VMEM((_ZB, REP), jnp.float32),
            pltpu.VMEM((16,), jnp.int32),
            pltpu.SemaphoreType.DMA,
        ],
    )
    return f(cycle_out, eid, rid, lanes, flags)


_RID_NP = np.concatenate(
    [np.repeat(np.arange(NCYC, dtype=np.int32) + i * NCYC, s)
     for i, s in enumerate(SIZES)])


def _scatter_c2e(cycle_out, idxs):
    eid = jnp.concatenate([idx.reshape(-1) for idx in idxs])
    eid = jnp.concatenate(
        [eid, jnp.full((_NPADI - _ITEMS,), np.int32(_NEP), jnp.int32)])
    rid = jnp.concatenate(
        [jnp.asarray(_RID_NP), jnp.zeros((_NPADI - _ITEMS,), jnp.int32)])
    lanes = jnp.arange(16, dtype=jnp.int32)
    ch = eid // _CS
    ch = ch.reshape(_NT, _GPT, 16)
    flags = (ch[:, :, :, None] == jnp.arange(_NCH, dtype=jnp.int32)) \
        .any(axis=2)
    flags = flags.transpose(0, 2, 1).astype(jnp.int32).reshape(-1)
    return _sc_scatter(cycle_out, eid, rid, lanes, flags)[:NE]


def kernel(edge_rep, cycle_rep, cyc3_idx, cyc4_idx, cyc5_idx, cyc6_idx, cyc7_idx, cyc8_idx,
           aut_W, cyc_W1, cyc_g1, cyc_b1, cyc_W2, cyc_g2, cyc_b2,
           edge_W1, edge_g1, edge_b1, edge_W2, edge_g2, edge_b2):
    idxs = [cyc3_idx, cyc4_idx, cyc5_idx, cyc6_idx, cyc7_idx, cyc8_idx]

    A = cyc_W1[:REP]
    Bs = cyc_W1[REP:].reshape(2, REP, 2 * REP)
    Ms = jnp.einsum('cikl,clo->iko', aut_W, Bs)

    pooled = _gather_pooled(edge_rep, idxs)
    cycle_out = _cycle_mlp(cycle_rep, pooled, A, Ms, cyc_g1, cyc_b1,
                           cyc_W2, cyc_g2, cyc_b2)

    c2e = _scatter_c2e(cycle_out, idxs)
    W1a = edge_W1[:REP]
    W1b = edge_W1[REP:]
    edge_out = _edge_mlp(edge_rep, c2e, W1a, W1b, edge_g1, edge_b1,
                         edge_W2, edge_g2, edge_b2)
    return edge_out, cycle_out


# speedup vs baseline: 2.8737x; 2.8737x over previous
import functools

import jax
import jax.numpy as jnp
import numpy as np
from jax import lax
from jax.experimental import pallas as pl
from jax.experimental.pallas import tpu as pltpu
from jax.experimental.pallas import tpu_sc as plsc

REP = 128
NE = 320000
NCYC = 5000
SIZES = (3, 4, 5, 6, 7, 8)
TOTC = NCYC * len(SIZES)
EPS = 1e-5


def _cyc_pass1_body(cr_ref, pooled_ref, a_ref, m_ref, stat_ref):
    i = pl.program_id(0)
    y1 = (jnp.dot(cr_ref[...], a_ref[...], preferred_element_type=jnp.float32)
          + jnp.dot(pooled_ref[...], m_ref[0], preferred_element_type=jnp.float32))

    @pl.when(i == 0)
    def _():
        stat_ref[...] = jnp.zeros_like(stat_ref)

    stat_ref[0, :] += jnp.sum(y1, axis=0)
    stat_ref[1, :] += jnp.sum(y1 * y1, axis=0)


def _cyc_pass2_body(cr_ref, pooled_ref, a_ref, m_ref, sc1_ref, w2_ref,
                    y2_ref, stat_ref):
    i = pl.program_id(0)
    y1 = (jnp.dot(cr_ref[...], a_ref[...], preferred_element_type=jnp.float32)
          + jnp.dot(pooled_ref[...], m_ref[0], preferred_element_type=jnp.float32))
    h = jnp.maximum(y1 * sc1_ref[0, :] + sc1_ref[1, :], 0.0)
    y2 = jnp.dot(h, w2_ref[...], preferred_element_type=jnp.float32)
    y2_ref[...] = y2

    @pl.when(i == 0)
    def _():
        stat_ref[...] = jnp.zeros_like(stat_ref)

    stat_ref[0, :] += jnp.sum(y2, axis=0)
    stat_ref[1, :] += jnp.sum(y2 * y2, axis=0)


def _norm_body(y_ref, sc_ref, out_ref):
    out_ref[...] = jnp.maximum(y_ref[...] * sc_ref[0, :] + sc_ref[1, :], 0.0)


def _edge_pass1_body(er_ref, c2e_ref, w1a_ref, w1b_ref, stat_ref):
    i = pl.program_id(0)
    y1 = (jnp.dot(er_ref[...], w1a_ref[...], preferred_element_type=jnp.float32)
          + jnp.dot(c2e_ref[...], w1b_ref[...], preferred_element_type=jnp.float32))

    @pl.when(i == 0)
    def _():
        stat_ref[...] = jnp.zeros_like(stat_ref)

    stat_ref[0, :] += jnp.sum(y1, axis=0)
    stat_ref[1, :] += jnp.sum(y1 * y1, axis=0)


def _edge_pass2_body(er_ref, c2e_ref, w1a_ref, w1b_ref, sc1_ref, w2_ref,
                     y2_ref, stat_ref):
    i = pl.program_id(0)
    y1 = (jnp.dot(er_ref[...], w1a_ref[...], preferred_element_type=jnp.float32)
          + jnp.dot(c2e_ref[...], w1b_ref[...], preferred_element_type=jnp.float32))
    h = jnp.maximum(y1 * sc1_ref[0, :] + sc1_ref[1, :], 0.0)
    y2 = jnp.dot(h, w2_ref[...], preferred_element_type=jnp.float32)
    y2_ref[...] = y2

    @pl.when(i == 0)
    def _():
        stat_ref[...] = jnp.zeros_like(stat_ref)

    stat_ref[0, :] += jnp.sum(y2, axis=0)
    stat_ref[1, :] += jnp.sum(y2 * y2, axis=0)


def _scale_shift(stats, n, g, b):
    mean = stats[0] / n
    var = stats[1] / n - mean * mean
    scale = g / jnp.sqrt(var + EPS)
    shift = b - mean * scale
    return jnp.stack([scale, shift])


_CB = 1000
_CG = TOTC // _CB
_EB = 2000
_EG = NE // _EB


def _row_spec(blk, width):
    return pl.BlockSpec((blk, width), lambda i: (i, 0))


def _full_spec(shape):
    return pl.BlockSpec(shape, lambda i: tuple(0 for _ in shape))


def _cycle_mlp(cycle_rep, pooled, A, Ms, cyc_g1, cyc_b1, cyc_W2, cyc_g2, cyc_b2):
    m_spec = pl.BlockSpec((1, REP, 2 * REP), lambda i: (i // (NCYC // _CB), 0, 0))
    stats1 = pl.pallas_call(
        _cyc_pass1_body,
        grid=(_CG,),
        in_specs=[_row_spec(_CB, REP), _row_spec(_CB, REP),
                  _full_spec((REP, 2 * REP)), m_spec],
        out_specs=_full_spec((2, 2 * REP)),
        out_shape=jax.ShapeDtypeStruct((2, 2 * REP), jnp.float32),
    )(cycle_rep, pooled, A, Ms)
    sc1 = _scale_shift(stats1, TOTC, cyc_g1, cyc_b1)

    y2, stats2 = pl.pallas_call(
        _cyc_pass2_body,
        grid=(_CG,),
        in_specs=[_row_spec(_CB, REP), _row_spec(_CB, REP),
                  _full_spec((REP, 2 * REP)), m_spec,
                  _full_spec((2, 2 * REP)), _full_spec((2 * REP, REP))],
        out_specs=[_row_spec(_CB, REP), _full_spec((2, REP))],
        out_shape=[jax.ShapeDtypeStruct((TOTC, REP), jnp.float32),
                   jax.ShapeDtypeStruct((2, REP), jnp.float32)],
    )(cycle_rep, pooled, A, Ms, sc1, cyc_W2)
    sc2 = _scale_shift(stats2, TOTC, cyc_g2, cyc_b2)

    cycle_out = pl.pallas_call(
        _norm_body,
        grid=(_CG,),
        in_specs=[_row_spec(_CB, REP), _full_spec((2, REP))],
        out_specs=_row_spec(_CB, REP),
        out_shape=jax.ShapeDtypeStruct((TOTC, REP), jnp.float32),
    )(y2, sc2)
    return cycle_out


def _edge_mlp(edge_rep, c2e, W1a, W1b, edge_g1, edge_b1, edge_W2, edge_g2, edge_b2):
    stats1 = pl.pallas_call(
        _edge_pass1_body,
        grid=(_EG,),
        in_specs=[_row_spec(_EB, REP), _row_spec(_EB, REP),
                  _full_spec((REP, 2 * REP)), _full_spec((REP, 2 * REP))],
        out_specs=_full_spec((2, 2 * REP)),
        out_shape=jax.ShapeDtypeStruct((2, 2 * REP), jnp.float32),
    )(edge_rep, c2e, W1a, W1b)
    sc1 = _scale_shift(stats1, NE, edge_g1, edge_b1)

    y2, stats2 = pl.pallas_call(
        _edge_pass2_body,
        grid=(_EG,),
        in_specs=[_row_spec(_EB, REP), _row_spec(_EB, REP),
                  _full_spec((REP, 2 * REP)), _full_spec((REP, 2 * REP)),
                  _full_spec((2, 2 * REP)), _full_spec((2 * REP, REP))],
        out_specs=[_row_spec(_EB, REP), _full_spec((2, REP))],
        out_shape=[jax.ShapeDtypeStruct((NE, REP), jnp.float32),
                   jax.ShapeDtypeStruct((2, REP), jnp.float32)],
    )(edge_rep, c2e, W1a, W1b, sc1, edge_W2)
    sc2 = _scale_shift(stats2, NE, edge_g2, edge_b2)

    edge_out = pl.pallas_call(
        _norm_body,
        grid=(_EG,),
        in_specs=[_row_spec(_EB, REP), _full_spec((2, REP))],
        out_specs=_row_spec(_EB, REP),
        out_shape=jax.ShapeDtypeStruct((NE, REP), jnp.float32),
    )(y2, sc2)
    return edge_out


def _gather_pooled(edge_rep, idxs):
    per_size = [jnp.take(edge_rep, idx, axis=0).sum(axis=1) for idx in idxs]
    return jnp.concatenate(per_size, axis=0)


def _scatter_c2e(cycle_out, idxs):
    c2e = jnp.zeros((NE, REP), dtype=cycle_out.dtype)
    off = 0
    for idx in idxs:
        n, sz = idx.shape
        co = cycle_out[off:off + n]
        c2e = c2e.at[idx.reshape(-1)].add(jnp.repeat(co, sz, axis=0))
        off += n
    return c2e


def kernel(edge_rep, cycle_rep, cyc3_idx, cyc4_idx, cyc5_idx, cyc6_idx, cyc7_idx, cyc8_idx,
           aut_W, cyc_W1, cyc_g1, cyc_b1, cyc_W2, cyc_g2, cyc_b2,
           edge_W1, edge_g1, edge_b1, edge_W2, edge_g2, edge_b2):
    idxs = [cyc3_idx, cyc4_idx, cyc5_idx, cyc6_idx, cyc7_idx, cyc8_idx]

    A = cyc_W1[:REP]
    Bs = cyc_W1[REP:].reshape(2, REP, 2 * REP)
    Ms = jnp.einsum('cikl,clo->iko', aut_W, Bs)

    pooled = _gather_pooled(edge_rep, idxs)
    cycle_out = _cycle_mlp(cycle_rep, pooled, A, Ms, cyc_g1, cyc_b1,
                           cyc_W2, cyc_g2, cyc_b2)

    c2e = _scatter_c2e(cycle_out, idxs)
    W1a = edge_W1[:REP]
    W1b = edge_W1[REP:]
    edge_out = _edge_mlp(edge_rep, c2e, W1a, W1b, edge_g1, edge_b1,
                         edge_W2, edge_g2, edge_b2)
    return edge_out, cycle_out


# speedup vs baseline: 3.1632x; 1.1007x over previous
import functools

import jax
import jax.numpy as jnp
import numpy as np
from jax import lax
from jax.experimental import pallas as pl
from jax.experimental.pallas import tpu as pltpu
from jax.experimental.pallas import tpu_sc as plsc

REP = 128
NE = 320000
NCYC = 5000
SIZES = (3, 4, 5, 6, 7, 8)
TOTC = NCYC * len(SIZES)
EPS = 1e-5


def _cyc_pass1_body(cr_ref, pooled_ref, a_ref, m_ref, stat_ref):
    i = pl.program_id(0)
    y1 = (jnp.dot(cr_ref[...], a_ref[...], preferred_element_type=jnp.float32)
          + jnp.dot(pooled_ref[...], m_ref[0], preferred_element_type=jnp.float32))

    @pl.when(i == 0)
    def _():
        stat_ref[...] = jnp.zeros_like(stat_ref)

    stat_ref[0, :] += jnp.sum(y1, axis=0)
    stat_ref[1, :] += jnp.sum(y1 * y1, axis=0)


def _cyc_pass2_body(cr_ref, pooled_ref, a_ref, m_ref, sc1_ref, w2_ref,
                    y2_ref, stat_ref):
    i = pl.program_id(0)
    y1 = (jnp.dot(cr_ref[...], a_ref[...], preferred_element_type=jnp.float32)
          + jnp.dot(pooled_ref[...], m_ref[0], preferred_element_type=jnp.float32))
    h = jnp.maximum(y1 * sc1_ref[0, :] + sc1_ref[1, :], 0.0)
    y2 = jnp.dot(h, w2_ref[...], preferred_element_type=jnp.float32)
    y2_ref[...] = y2

    @pl.when(i == 0)
    def _():
        stat_ref[...] = jnp.zeros_like(stat_ref)

    stat_ref[0, :] += jnp.sum(y2, axis=0)
    stat_ref[1, :] += jnp.sum(y2 * y2, axis=0)


def _norm_body(y_ref, sc_ref, out_ref):
    out_ref[...] = jnp.maximum(y_ref[...] * sc_ref[0, :] + sc_ref[1, :], 0.0)


def _edge_pass1_body(er_ref, c2e_ref, w1a_ref, w1b_ref, stat_ref):
    i = pl.program_id(0)
    y1 = (jnp.dot(er_ref[...], w1a_ref[...], preferred_element_type=jnp.float32)
          + jnp.dot(c2e_ref[...], w1b_ref[...], preferred_element_type=jnp.float32))

    @pl.when(i == 0)
    def _():
        stat_ref[...] = jnp.zeros_like(stat_ref)

    stat_ref[0, :] += jnp.sum(y1, axis=0)
    stat_ref[1, :] += jnp.sum(y1 * y1, axis=0)


def _edge_pass2_body(er_ref, c2e_ref, w1a_ref, w1b_ref, sc1_ref, w2_ref,
                     y2_ref, stat_ref):
    i = pl.program_id(0)
    y1 = (jnp.dot(er_ref[...], w1a_ref[...], preferred_element_type=jnp.float32)
          + jnp.dot(c2e_ref[...], w1b_ref[...], preferred_element_type=jnp.float32))
    h = jnp.maximum(y1 * sc1_ref[0, :] + sc1_ref[1, :], 0.0)
    y2 = jnp.dot(h, w2_ref[...], preferred_element_type=jnp.float32)
    y2_ref[...] = y2

    @pl.when(i == 0)
    def _():
        stat_ref[...] = jnp.zeros_like(stat_ref)

    stat_ref[0, :] += jnp.sum(y2, axis=0)
    stat_ref[1, :] += jnp.sum(y2 * y2, axis=0)


def _scale_shift(stats, n, g, b):
    mean = stats[0] / n
    var = stats[1] / n - mean * mean
    scale = g / jnp.sqrt(var + EPS)
    shift = b - mean * scale
    return jnp.stack([scale, shift])


_CB = 1000
_CG = TOTC // _CB
_EB = 8000
_EG = NE // _EB


def _row_spec(blk, width):
    return pl.BlockSpec((blk, width), lambda i: (i, 0))


def _full_spec(shape):
    return pl.BlockSpec(shape, lambda i: tuple(0 for _ in shape))


def _cycle_mlp(cycle_rep, pooled, A, Ms, cyc_g1, cyc_b1, cyc_W2, cyc_g2, cyc_b2):
    m_spec = pl.BlockSpec((1, REP, 2 * REP), lambda i: (i // (NCYC // _CB), 0, 0))
    stats1 = pl.pallas_call(
        _cyc_pass1_body,
        grid=(_CG,),
        in_specs=[_row_spec(_CB, REP), _row_spec(_CB, REP),
                  _full_spec((REP, 2 * REP)), m_spec],
        out_specs=_full_spec((2, 2 * REP)),
        out_shape=jax.ShapeDtypeStruct((2, 2 * REP), jnp.float32),
    )(cycle_rep, pooled, A, Ms)
    sc1 = _scale_shift(stats1, TOTC, cyc_g1, cyc_b1)

    y2, stats2 = pl.pallas_call(
        _cyc_pass2_body,
        grid=(_CG,),
        in_specs=[_row_spec(_CB, REP), _row_spec(_CB, REP),
                  _full_spec((REP, 2 * REP)), m_spec,
                  _full_spec((2, 2 * REP)), _full_spec((2 * REP, REP))],
        out_specs=[_row_spec(_CB, REP), _full_spec((2, REP))],
        out_shape=[jax.ShapeDtypeStruct((TOTC, REP), jnp.float32),
                   jax.ShapeDtypeStruct((2, REP), jnp.float32)],
    )(cycle_rep, pooled, A, Ms, sc1, cyc_W2)
    sc2 = _scale_shift(stats2, TOTC, cyc_g2, cyc_b2)

    cycle_out = pl.pallas_call(
        _norm_body,
        grid=(_CG,),
        in_specs=[_row_spec(_CB, REP), _full_spec((2, REP))],
        out_specs=_row_spec(_CB, REP),
        out_shape=jax.ShapeDtypeStruct((TOTC, REP), jnp.float32),
    )(y2, sc2)
    return cycle_out


def _edge_mlp(edge_rep, c2e, W1a, W1b, edge_g1, edge_b1, edge_W2, edge_g2, edge_b2):
    stats1 = pl.pallas_call(
        _edge_pass1_body,
        grid=(_EG,),
        in_specs=[_row_spec(_EB, REP), _row_spec(_EB, REP),
                  _full_spec((REP, 2 * REP)), _full_spec((REP, 2 * REP))],
        out_specs=_full_spec((2, 2 * REP)),
        out_shape=jax.ShapeDtypeStruct((2, 2 * REP), jnp.float32),
    )(edge_rep, c2e, W1a, W1b)
    sc1 = _scale_shift(stats1, NE, edge_g1, edge_b1)

    y2, stats2 = pl.pallas_call(
        _edge_pass2_body,
        grid=(_EG,),
        in_specs=[_row_spec(_EB, REP), _row_spec(_EB, REP),
                  _full_spec((REP, 2 * REP)), _full_spec((REP, 2 * REP)),
                  _full_spec((2, 2 * REP)), _full_spec((2 * REP, REP))],
        out_specs=[_row_spec(_EB, REP), _full_spec((2, REP))],
        out_shape=[jax.ShapeDtypeStruct((NE, REP), jnp.float32),
                   jax.ShapeDtypeStruct((2, REP), jnp.float32)],
    )(edge_rep, c2e, W1a, W1b, sc1, edge_W2)
    sc2 = _scale_shift(stats2, NE, edge_g2, edge_b2)

    edge_out = pl.pallas_call(
        _norm_body,
        grid=(_EG,),
        in_specs=[_row_spec(_EB, REP), _full_spec((2, REP))],
        out_specs=_row_spec(_EB, REP),
        out_shape=jax.ShapeDtypeStruct((NE, REP), jnp.float32),
    )(y2, sc2)
    return edge_out


def _gather_pooled(edge_rep, idxs):
    per_size = [jnp.take(edge_rep, idx, axis=0).sum(axis=1) for idx in idxs]
    return jnp.concatenate(per_size, axis=0)


def _scatter_c2e(cycle_out, idxs):
    c2e = jnp.zeros((NE, REP), dtype=cycle_out.dtype)
    off = 0
    for idx in idxs:
        n, sz = idx.shape
        co = cycle_out[off:off + n]
        c2e = c2e.at[idx.reshape(-1)].add(jnp.repeat(co, sz, axis=0))
        off += n
    return c2e


def kernel(edge_rep, cycle_rep, cyc3_idx, cyc4_idx, cyc5_idx, cyc6_idx, cyc7_idx, cyc8_idx,
           aut_W, cyc_W1, cyc_g1, cyc_b1, cyc_W2, cyc_g2, cyc_b2,
           edge_W1, edge_g1, edge_b1, edge_W2, edge_g2, edge_b2):
    idxs = [cyc3_idx, cyc4_idx, cyc5_idx, cyc6_idx, cyc7_idx, cyc8_idx]

    A = cyc_W1[:REP]
    Bs = cyc_W1[REP:].reshape(2, REP, 2 * REP)
    Ms = jnp.einsum('cikl,clo->iko', aut_W, Bs)

    pooled = _gather_pooled(edge_rep, idxs)
    cycle_out = _cycle_mlp(cycle_rep, pooled, A, Ms, cyc_g1, cyc_b1,
                           cyc_W2, cyc_g2, cyc_b2)

    c2e = _scatter_c2e(cycle_out, idxs)
    W1a = edge_W1[:REP]
    W1b = edge_W1[REP:]
    edge_out = _edge_mlp(edge_rep, c2e, W1a, W1b, edge_g1, edge_b1,
                         edge_W2, edge_g2, edge_b2)
    return edge_out, cycle_out
